# baseline (speedup 1.0000x reference)
import functools

import jax
import jax.numpy as jnp
from jax import lax
from jax.experimental import pallas as pl
from jax.experimental.pallas import tpu as pltpu
from jax.experimental.pallas import tpu_sc as plsc

_CH = 64
_EPS = 1e-5
_L = 16
_NC = 2
_NS = 16
_NW = _NC * _NS
_BB = 128


def _rsqrt(t):
    i = lax.bitcast_convert_type(t, jnp.int32)
    i = jnp.int32(0x5F3759DF) - (i >> 1)
    y = lax.bitcast_convert_type(i, jnp.float32)
    for _ in range(3):
        y = y * (1.5 - 0.5 * t * y * y)
    return y


def _body(n_m, xt_hbm, table_hbm, gamma_hbm, beta_hbm, out_hbm,
          idx_v, pidx0_v, pidx1_v, in0_v, in1_v, tr0_v, tr1_v,
          gam_v, bet_v, gsem0, gsem1, ssem0, ssem1):
    cid = lax.axis_index("c")
    sid = lax.axis_index("s")
    wid = sid * _NC + cid

    pltpu.sync_copy(xt_hbm.at[:, pl.ds(wid * _BB, _BB)], idx_v)
    pltpu.sync_copy(gamma_hbm, gam_v)
    pltpu.sync_copy(beta_hbm, bet_v)

    ga = []
    be = []
    for k in range(_CH // _L):
        gk = gam_v[pl.ds(k * _L, _L)]
        bk = bet_v[pl.ds(k * _L, _L)]
        for l in range(_L):
            ga.append(gk[l])
            be.append(bk[l])

    inb = (in0_v, in1_v)
    trb = (tr0_v, tr1_v)
    pidx = (pidx0_v, pidx1_v)
    gsem = (gsem0, gsem1)
    ssem = (ssem0, ssem1)

    def start_gather(m, b):
        for g in range(_BB // _L):
            raw = idx_v[m, pl.ds(g * _L, _L)]
            pidx[b][pl.ds(g * _L, _L)] = raw >> 1
        pltpu.make_async_copy(table_hbm.at[pidx[b]], inb[b], gsem[b]).start()

    def wait_gather(b):
        pltpu.make_async_copy(table_hbm.at[pidx[b]], inb[b], gsem[b]).wait()

    def start_store(m, b):
        pltpu.make_async_copy(trb[b], out_hbm.at[m, :, wid], ssem[b]).start()

    def wait_store(b):
        pltpu.make_async_copy(trb[b], out_hbm.at[0, :, wid], ssem[b]).wait()

    start_gather(0, 0)
    start_gather(1, 1)

    @pl.loop(0, n_m, step=2)
    def _(mm):
        for b in range(2):
            m = mm + b
            wait_gather(b)

            @pl.when(m >= 2)
            def _():
                wait_store(b)

            src, dst = inb[b], trb[b]

            @plsc.parallel_loop(0, _BB, _L)
            def _(r0):
                raw = idx_v[m, pl.ds(r0, _L)]
                rows = r0 + lax.iota(jnp.int32, _L)
                cbase = (raw & 1) * _CH
                s = jnp.zeros((_L,), jnp.float32)
                q = jnp.zeros((_L,), jnp.float32)
                for c in range(_CH):
                    v = plsc.load_gather(src, [rows, cbase + c])
                    s = s + v
                    q = q + v * v
                    dst[c // 8, c % 8, pl.ds(r0, _L)] = v
                mean = s * (1.0 / _CH)
                var = q * (1.0 / _CH) - mean * mean
                a = _rsqrt(var + _EPS)
                for c in range(_CH):
                    ac = a * ga[c]
                    bc = be[c] - mean * ac
                    sl = dst.at[c // 8, c % 8, pl.ds(r0, _L)]
                    sl[...] = sl[...] * ac + bc

            start_store(m, b)

            @pl.when(m + 2 < n_m)
            def _():
                start_gather(m + 2, b)

    wait_store(0)
    wait_store(1)


def kernel(x, table, gamma, beta):
    n_b, n_m = x.shape
    xt = x.T
    table2 = table.reshape(table.shape[0] // 2, 2 * _CH)

    mesh = plsc.VectorSubcoreMesh(
        core_axis_name="c", subcore_axis_name="s",
        num_cores=_NC, num_subcores=_NS)

    run = pl.kernel(
        functools.partial(_body, n_m),
        out_type=jax.ShapeDtypeStruct((n_m, 8, _NW, 8, _BB), jnp.float32),
        mesh=mesh,
        compiler_params=pltpu.CompilerParams(
            needs_layout_passes=False, use_tc_tiling_on_sc=False),
        scratch_types=[
            pltpu.VMEM((n_m, _BB), jnp.int32),
            pltpu.VMEM((_BB,), jnp.int32),
            pltpu.VMEM((_BB,), jnp.int32),
            pltpu.VMEM((_BB, 128), jnp.float32),
            pltpu.VMEM((_BB, 128), jnp.float32),
            pltpu.VMEM((8, 8, _BB), jnp.float32),
            pltpu.VMEM((8, 8, _BB), jnp.float32),
            pltpu.VMEM((_CH,), jnp.float32),
            pltpu.VMEM((_CH,), jnp.float32),
            pltpu.SemaphoreType.DMA,
            pltpu.SemaphoreType.DMA,
            pltpu.SemaphoreType.DMA,
            pltpu.SemaphoreType.DMA,
        ],
    )
    out5d = run(xt, table2, gamma, beta)
    out = out5d.transpose(2, 4, 0, 1, 3).reshape(n_b, n_m, _CH)
    return out
